# baseline (speedup 1.0000x reference)
import functools

import jax
import jax.numpy as jnp
from jax import lax
from jax.experimental import pallas as pl
from jax.experimental.pallas import tpu as pltpu
from jax.experimental.pallas import tpu_sc as plsc

VOCAB = 100000
DIM = 16
BATCH = 1024


_INFO = plsc.get_sparse_core_info()
_NC, _NS = _INFO.num_cores, _INFO.num_subcores
_NW = _NC * _NS
_B2 = 2 * BATCH
_BPW = _B2 // _NW
_HALF_W = _NW // 2


@functools.partial(
    pl.kernel,
    mesh=plsc.VectorSubcoreMesh(core_axis_name="c", subcore_axis_name="s"),
    out_type=jax.ShapeDtypeStruct((_B2, DIM), jnp.float32),
    scratch_types=[
        pltpu.VMEM((_BPW,), jnp.int32),
        pltpu.VMEM((_BPW, DIM), jnp.float32),
        pltpu.SemaphoreType.DMA,
    ],
    compiler_params=pltpu.CompilerParams(use_tc_tiling_on_sc=False),
)
def _sc_gather(table_hbm, i_hbm, j_hbm, out_hbm, idx_v, rows_v, sem):
    wid = lax.axis_index("s") * _NC + lax.axis_index("c")

    @pl.when(wid < _HALF_W)
    def _load_i():
        pltpu.sync_copy(i_hbm.at[pl.ds(wid * _BPW, _BPW)], idx_v)

    @pl.when(wid >= _HALF_W)
    def _load_j():
        pltpu.sync_copy(j_hbm.at[pl.ds((wid - _HALF_W) * _BPW, _BPW)], idx_v)

    pltpu.async_copy(table_hbm.at[idx_v], rows_v, sem).wait()
    pltpu.sync_copy(rows_v, out_hbm.at[pl.ds(wid * _BPW, _BPW)])


_VT = 4096
_NT = (VOCAB + _VT - 1) // _VT


def _tr_body(w1_ref, w2_ref, o1_ref, o2_ref):
    o1_ref[...] = w1_ref[...].T
    o2_ref[...] = w2_ref[...].T


def _tc_transpose(W1, W2):
    return pl.pallas_call(
        _tr_body,
        grid=(_NT,),
        in_specs=[
            pl.BlockSpec((_VT, DIM), lambda t: (t, 0)),
            pl.BlockSpec((_VT, DIM), lambda t: (t, 0)),
        ],
        out_specs=[
            pl.BlockSpec((DIM, _VT), lambda t: (0, t)),
            pl.BlockSpec((DIM, _VT), lambda t: (0, t)),
        ],
        out_shape=[
            jax.ShapeDtypeStruct((DIM, VOCAB), jnp.float32),
            jax.ShapeDtypeStruct((DIM, VOCAB), jnp.float32),
        ],
        compiler_params=pltpu.CompilerParams(
            dimension_semantics=("parallel",),
        ),
    )(W1, W2)


_RB = 16
_NSTEPS = BATCH // _RB
_NBUF = 2
_VBULK = (VOCAB // 128) * 128
_VTAIL = VOCAB - _VBULK


def _mm_body(xi_ref, xj_ref, w1t_ref, b1_ref, w2t_ref, b2_ref,
             o1_hbm, o2_hbm, o1_buf, o2_buf, t1_buf, t2_buf, sem1, sem2,
             tsem):
    i = pl.program_id(0)
    nb = lax.rem(i, _NBUF)
    dn = (((1,), (0,)), ((), ()))

    @pl.when(i >= _NBUF)
    def _drain_oldest():
        j = i - _NBUF
        pltpu.make_async_copy(
            o1_buf.at[nb, :, pl.ds(0, _VBULK)],
            o1_hbm.at[pl.ds(j * _RB, _RB), pl.ds(0, _VBULK)],
            sem1.at[nb]).wait()
        pltpu.make_async_copy(
            o2_buf.at[nb, :, pl.ds(0, _VBULK)],
            o2_hbm.at[pl.ds(j * _RB, _RB), pl.ds(0, _VBULK)],
            sem2.at[nb]).wait()

    o1_buf[nb] = (
        lax.dot_general(xi_ref[...], w1t_ref[...], dn,
                        preferred_element_type=jnp.float32)
        + b1_ref[...][None, :]
    )
    o2_buf[nb] = (
        lax.dot_general(xj_ref[...], w2t_ref[...], dn,
                        preferred_element_type=jnp.float32)
        + b2_ref[...][None, :]
    )
    t1_buf[pl.ds(i * _RB, _RB), :] = o1_buf[nb, :, pl.ds(_VBULK, _VTAIL)]
    t2_buf[pl.ds(i * _RB, _RB), :] = o2_buf[nb, :, pl.ds(_VBULK, _VTAIL)]

    pltpu.make_async_copy(
        o1_buf.at[nb, :, pl.ds(0, _VBULK)],
        o1_hbm.at[pl.ds(i * _RB, _RB), pl.ds(0, _VBULK)],
        sem1.at[nb]).start()
    pltpu.make_async_copy(
        o2_buf.at[nb, :, pl.ds(0, _VBULK)],
        o2_hbm.at[pl.ds(i * _RB, _RB), pl.ds(0, _VBULK)],
        sem2.at[nb]).start()

    @pl.when(i == _NSTEPS - 1)
    def _drain_all():
        for k in range(_NBUF):
            j = _NSTEPS - 1 - k
            b = lax.rem(jnp.int32(j), _NBUF)
            pltpu.make_async_copy(
                o1_buf.at[b, :, pl.ds(0, _VBULK)],
                o1_hbm.at[pl.ds(j * _RB, _RB), pl.ds(0, _VBULK)],
                sem1.at[b]).wait()
            pltpu.make_async_copy(
                o2_buf.at[b, :, pl.ds(0, _VBULK)],
                o2_hbm.at[pl.ds(j * _RB, _RB), pl.ds(0, _VBULK)],
                sem2.at[b]).wait()
        c1 = pltpu.make_async_copy(
            t1_buf, o1_hbm.at[:, pl.ds(_VBULK, _VTAIL)], tsem)
        c1.start()
        c2 = pltpu.make_async_copy(
            t2_buf, o2_hbm.at[:, pl.ds(_VBULK, _VTAIL)], tsem)
        c2.start()
        c1.wait()
        c2.wait()


def _tc_matmuls(rows, W1t, b1, W2t, b2):
    nblk = BATCH // _RB
    return pl.pallas_call(
        _mm_body,
        grid=(_NSTEPS,),
        in_specs=[
            pl.BlockSpec((_RB, DIM), lambda v: (v, 0)),
            pl.BlockSpec((_RB, DIM), lambda v: (v + nblk, 0)),
            pl.BlockSpec((DIM, VOCAB), lambda v: (0, 0)),
            pl.BlockSpec((VOCAB,), lambda v: (0,)),
            pl.BlockSpec((DIM, VOCAB), lambda v: (0, 0)),
            pl.BlockSpec((VOCAB,), lambda v: (0,)),
        ],
        out_specs=[
            pl.BlockSpec(memory_space=pl.ANY),
            pl.BlockSpec(memory_space=pl.ANY),
        ],
        out_shape=[
            jax.ShapeDtypeStruct((BATCH, VOCAB), jnp.float32),
            jax.ShapeDtypeStruct((BATCH, VOCAB), jnp.float32),
        ],
        scratch_shapes=[
            pltpu.VMEM((_NBUF, _RB, VOCAB), jnp.float32),
            pltpu.VMEM((_NBUF, _RB, VOCAB), jnp.float32),
            pltpu.VMEM((BATCH, _VTAIL), jnp.float32),
            pltpu.VMEM((BATCH, _VTAIL), jnp.float32),
            pltpu.SemaphoreType.DMA((_NBUF,)),
            pltpu.SemaphoreType.DMA((_NBUF,)),
            pltpu.SemaphoreType.DMA,
        ],
        compiler_params=pltpu.CompilerParams(
            dimension_semantics=("arbitrary",),
        ),
    )(rows, rows, W1t, b1, W2t, b2)


def kernel(i_indices, j_indices, emb, W1, b1, W2, b2):
    rows = _sc_gather(
        emb, i_indices.astype(jnp.int32), j_indices.astype(jnp.int32)
    )
    W1t, W2t = _tc_transpose(W1, W2)
    return _tc_matmuls(rows, W1t, b1, W2t, b2)


# speedup vs baseline: 1.0072x; 1.0072x over previous
import functools

import jax
import jax.numpy as jnp
from jax import lax
from jax.experimental import pallas as pl
from jax.experimental.pallas import tpu as pltpu
from jax.experimental.pallas import tpu_sc as plsc

VOCAB = 100000
DIM = 16
BATCH = 1024


_INFO = plsc.get_sparse_core_info()
_NC, _NS = _INFO.num_cores, _INFO.num_subcores
_NW = _NC * _NS
_B2 = 2 * BATCH
_BPW = _B2 // _NW
_HALF_W = _NW // 2


@functools.partial(
    pl.kernel,
    mesh=plsc.VectorSubcoreMesh(core_axis_name="c", subcore_axis_name="s"),
    out_type=jax.ShapeDtypeStruct((_B2, DIM), jnp.float32),
    scratch_types=[
        pltpu.VMEM((_BPW,), jnp.int32),
        pltpu.VMEM((_BPW, DIM), jnp.float32),
        pltpu.SemaphoreType.DMA,
    ],
    compiler_params=pltpu.CompilerParams(use_tc_tiling_on_sc=False),
)
def _sc_gather(table_hbm, i_hbm, j_hbm, out_hbm, idx_v, rows_v, sem):
    wid = lax.axis_index("s") * _NC + lax.axis_index("c")

    @pl.when(wid < _HALF_W)
    def _load_i():
        pltpu.sync_copy(i_hbm.at[pl.ds(wid * _BPW, _BPW)], idx_v)

    @pl.when(wid >= _HALF_W)
    def _load_j():
        pltpu.sync_copy(j_hbm.at[pl.ds((wid - _HALF_W) * _BPW, _BPW)], idx_v)

    pltpu.async_copy(table_hbm.at[idx_v], rows_v, sem).wait()
    pltpu.sync_copy(rows_v, out_hbm.at[pl.ds(wid * _BPW, _BPW)])


_VT = 4096
_NT = (VOCAB + _VT - 1) // _VT


def _tr_body(w1_ref, w2_ref, o1_ref, o2_ref):
    e = jnp.eye(DIM, dtype=jnp.float32)
    dn = (((0,), (1,)), ((), ()))
    o1_ref[...] = lax.dot_general(e, w1_ref[...], dn,
                                  preferred_element_type=jnp.float32)
    o2_ref[...] = lax.dot_general(e, w2_ref[...], dn,
                                  preferred_element_type=jnp.float32)


def _tc_transpose(W1, W2):
    return pl.pallas_call(
        _tr_body,
        grid=(_NT,),
        in_specs=[
            pl.BlockSpec((_VT, DIM), lambda t: (t, 0)),
            pl.BlockSpec((_VT, DIM), lambda t: (t, 0)),
        ],
        out_specs=[
            pl.BlockSpec((DIM, _VT), lambda t: (0, t)),
            pl.BlockSpec((DIM, _VT), lambda t: (0, t)),
        ],
        out_shape=[
            jax.ShapeDtypeStruct((DIM, VOCAB), jnp.float32),
            jax.ShapeDtypeStruct((DIM, VOCAB), jnp.float32),
        ],
        compiler_params=pltpu.CompilerParams(
            dimension_semantics=("parallel",),
        ),
    )(W1, W2)


_RB = 16
_NSTEPS = BATCH // _RB
_NBUF = 2
_VBULK = (VOCAB // 128) * 128
_VTAIL = VOCAB - _VBULK


def _mm_body(xi_ref, xj_ref, w1t_ref, b1_ref, w2t_ref, b2_ref,
             o1_hbm, o2_hbm, o1_buf, o2_buf, t1_buf, t2_buf, sem1, sem2,
             tsem):
    i = pl.program_id(0)
    nb = lax.rem(i, _NBUF)
    dn = (((1,), (0,)), ((), ()))

    @pl.when(i >= _NBUF)
    def _drain_oldest():
        j = i - _NBUF
        pltpu.make_async_copy(
            o1_buf.at[nb, :, pl.ds(0, _VBULK)],
            o1_hbm.at[pl.ds(j * _RB, _RB), pl.ds(0, _VBULK)],
            sem1.at[nb]).wait()
        pltpu.make_async_copy(
            o2_buf.at[nb, :, pl.ds(0, _VBULK)],
            o2_hbm.at[pl.ds(j * _RB, _RB), pl.ds(0, _VBULK)],
            sem2.at[nb]).wait()

    o1_buf[nb] = (
        lax.dot_general(xi_ref[...], w1t_ref[...], dn,
                        preferred_element_type=jnp.float32)
        + b1_ref[...][None, :]
    )
    o2_buf[nb] = (
        lax.dot_general(xj_ref[...], w2t_ref[...], dn,
                        preferred_element_type=jnp.float32)
        + b2_ref[...][None, :]
    )
    t1_buf[pl.ds(i * _RB, _RB), :] = o1_buf[nb, :, pl.ds(_VBULK, _VTAIL)]
    t2_buf[pl.ds(i * _RB, _RB), :] = o2_buf[nb, :, pl.ds(_VBULK, _VTAIL)]

    pltpu.make_async_copy(
        o1_buf.at[nb, :, pl.ds(0, _VBULK)],
        o1_hbm.at[pl.ds(i * _RB, _RB), pl.ds(0, _VBULK)],
        sem1.at[nb]).start()
    pltpu.make_async_copy(
        o2_buf.at[nb, :, pl.ds(0, _VBULK)],
        o2_hbm.at[pl.ds(i * _RB, _RB), pl.ds(0, _VBULK)],
        sem2.at[nb]).start()

    @pl.when(i == _NSTEPS - 1)
    def _drain_all():
        for k in range(_NBUF):
            j = _NSTEPS - 1 - k
            b = lax.rem(jnp.int32(j), _NBUF)
            pltpu.make_async_copy(
                o1_buf.at[b, :, pl.ds(0, _VBULK)],
                o1_hbm.at[pl.ds(j * _RB, _RB), pl.ds(0, _VBULK)],
                sem1.at[b]).wait()
            pltpu.make_async_copy(
                o2_buf.at[b, :, pl.ds(0, _VBULK)],
                o2_hbm.at[pl.ds(j * _RB, _RB), pl.ds(0, _VBULK)],
                sem2.at[b]).wait()
        c1 = pltpu.make_async_copy(
            t1_buf, o1_hbm.at[:, pl.ds(_VBULK, _VTAIL)], tsem)
        c1.start()
        c2 = pltpu.make_async_copy(
            t2_buf, o2_hbm.at[:, pl.ds(_VBULK, _VTAIL)], tsem)
        c2.start()
        c1.wait()
        c2.wait()


def _tc_matmuls(rows, W1t, b1, W2t, b2):
    nblk = BATCH // _RB
    return pl.pallas_call(
        _mm_body,
        grid=(_NSTEPS,),
        in_specs=[
            pl.BlockSpec((_RB, DIM), lambda v: (v, 0)),
            pl.BlockSpec((_RB, DIM), lambda v: (v + nblk, 0)),
            pl.BlockSpec((DIM, VOCAB), lambda v: (0, 0)),
            pl.BlockSpec((VOCAB,), lambda v: (0,)),
            pl.BlockSpec((DIM, VOCAB), lambda v: (0, 0)),
            pl.BlockSpec((VOCAB,), lambda v: (0,)),
        ],
        out_specs=[
            pl.BlockSpec(memory_space=pl.ANY),
            pl.BlockSpec(memory_space=pl.ANY),
        ],
        out_shape=[
            jax.ShapeDtypeStruct((BATCH, VOCAB), jnp.float32),
            jax.ShapeDtypeStruct((BATCH, VOCAB), jnp.float32),
        ],
        scratch_shapes=[
            pltpu.VMEM((_NBUF, _RB, VOCAB), jnp.float32),
            pltpu.VMEM((_NBUF, _RB, VOCAB), jnp.float32),
            pltpu.VMEM((BATCH, _VTAIL), jnp.float32),
            pltpu.VMEM((BATCH, _VTAIL), jnp.float32),
            pltpu.SemaphoreType.DMA((_NBUF,)),
            pltpu.SemaphoreType.DMA((_NBUF,)),
            pltpu.SemaphoreType.DMA,
        ],
        compiler_params=pltpu.CompilerParams(
            dimension_semantics=("arbitrary",),
        ),
    )(rows, rows, W1t, b1, W2t, b2)


def kernel(i_indices, j_indices, emb, W1, b1, W2, b2):
    rows = _sc_gather(
        emb, i_indices.astype(jnp.int32), j_indices.astype(jnp.int32)
    )
    W1t, W2t = _tc_transpose(W1, W2)
    return _tc_matmuls(rows, W1t, b1, W2t, b2)


# speedup vs baseline: 3.4422x; 3.4175x over previous
import functools

import jax
import jax.numpy as jnp
from jax import lax
from jax.experimental import pallas as pl
from jax.experimental.pallas import tpu as pltpu
from jax.experimental.pallas import tpu_sc as plsc

VOCAB = 100000
DIM = 16
BATCH = 1024


_INFO = plsc.get_sparse_core_info()
_NC, _NS = _INFO.num_cores, _INFO.num_subcores
_NW = _NC * _NS
_B2 = 2 * BATCH
_BPW = _B2 // _NW
_HALF_W = _NW // 2


@functools.partial(
    pl.kernel,
    mesh=plsc.VectorSubcoreMesh(core_axis_name="c", subcore_axis_name="s"),
    out_type=jax.ShapeDtypeStruct((_B2, DIM), jnp.float32),
    scratch_types=[
        pltpu.VMEM((_BPW,), jnp.int32),
        pltpu.VMEM((_BPW, DIM), jnp.float32),
        pltpu.SemaphoreType.DMA,
    ],
    compiler_params=pltpu.CompilerParams(use_tc_tiling_on_sc=False),
)
def _sc_gather(table_hbm, i_hbm, j_hbm, out_hbm, idx_v, rows_v, sem):
    wid = lax.axis_index("s") * _NC + lax.axis_index("c")

    @pl.when(wid < _HALF_W)
    def _load_i():
        pltpu.sync_copy(i_hbm.at[pl.ds(wid * _BPW, _BPW)], idx_v)

    @pl.when(wid >= _HALF_W)
    def _load_j():
        pltpu.sync_copy(j_hbm.at[pl.ds((wid - _HALF_W) * _BPW, _BPW)], idx_v)

    pltpu.async_copy(table_hbm.at[idx_v], rows_v, sem).wait()
    pltpu.sync_copy(rows_v, out_hbm.at[pl.ds(wid * _BPW, _BPW)])


_KA = DIM + 1
_VB = 2048
_NSTEPS = (VOCAB + _VB - 1) // _VB


def _mm_body(xia_ref, xja_ref, w1a_ref, w2a_ref, o1_ref, o2_ref):
    dn = (((0,), (1,)), ((), ()))
    o1_ref[...] = lax.dot_general(
        w1a_ref[...], xia_ref[...], dn, preferred_element_type=jnp.float32)
    o2_ref[...] = lax.dot_general(
        w2a_ref[...], xja_ref[...], dn, preferred_element_type=jnp.float32)


def _tc_matmuls(xa, W1a, W2a):
    return pl.pallas_call(
        _mm_body,
        grid=(_NSTEPS,),
        in_specs=[
            pl.BlockSpec((BATCH, _KA), lambda v: (0, 0)),
            pl.BlockSpec((BATCH, _KA), lambda v: (1, 0)),
            pl.BlockSpec((_KA, _VB), lambda v: (0, v)),
            pl.BlockSpec((_KA, _VB), lambda v: (0, v)),
        ],
        out_specs=[
            pl.BlockSpec((_VB, BATCH), lambda v: (v, 0)),
            pl.BlockSpec((_VB, BATCH), lambda v: (v, 0)),
        ],
        out_shape=[
            jax.ShapeDtypeStruct((VOCAB, BATCH), jnp.float32),
            jax.ShapeDtypeStruct((VOCAB, BATCH), jnp.float32),
        ],
        compiler_params=pltpu.CompilerParams(
            dimension_semantics=("parallel",),
        ),
    )(xa, xa, W1a, W2a)


def kernel(i_indices, j_indices, emb, W1, b1, W2, b2):
    rows = _sc_gather(
        emb, i_indices.astype(jnp.int32), j_indices.astype(jnp.int32)
    )
    ones = jnp.ones((_B2, 1), dtype=jnp.float32)
    xa = jnp.concatenate([rows, ones], axis=1)
    W1a = jnp.concatenate([W1.T, b1[None, :]], axis=0)
    W2a = jnp.concatenate([W2.T, b2[None, :]], axis=0)
    o1T, o2T = _tc_matmuls(xa, W1a, W2a)
    return (o1T.T, o2T.T)


# speedup vs baseline: 3.5583x; 1.0337x over previous
import functools

import jax
import jax.numpy as jnp
from jax import lax
from jax.experimental import pallas as pl
from jax.experimental.pallas import tpu as pltpu
from jax.experimental.pallas import tpu_sc as plsc

VOCAB = 100000
DIM = 16
BATCH = 1024


_INFO = plsc.get_sparse_core_info()
_NC, _NS = _INFO.num_cores, _INFO.num_subcores
_NW = _NC * _NS
_B2 = 2 * BATCH
_BPW = _B2 // _NW
_HALF_W = _NW // 2


@functools.partial(
    pl.kernel,
    mesh=plsc.VectorSubcoreMesh(core_axis_name="c", subcore_axis_name="s"),
    out_type=jax.ShapeDtypeStruct((_B2, DIM), jnp.float32),
    scratch_types=[
        pltpu.VMEM((_BPW,), jnp.int32),
        pltpu.VMEM((_BPW, DIM), jnp.float32),
        pltpu.SemaphoreType.DMA,
    ],
    compiler_params=pltpu.CompilerParams(use_tc_tiling_on_sc=False),
)
def _sc_gather(table_hbm, i_hbm, j_hbm, out_hbm, idx_v, rows_v, sem):
    wid = lax.axis_index("s") * _NC + lax.axis_index("c")

    @pl.when(wid < _HALF_W)
    def _load_i():
        pltpu.sync_copy(i_hbm.at[pl.ds(wid * _BPW, _BPW)], idx_v)

    @pl.when(wid >= _HALF_W)
    def _load_j():
        pltpu.sync_copy(j_hbm.at[pl.ds((wid - _HALF_W) * _BPW, _BPW)], idx_v)

    pltpu.async_copy(table_hbm.at[idx_v], rows_v, sem).wait()
    pltpu.sync_copy(rows_v, out_hbm.at[pl.ds(wid * _BPW, _BPW)])


_VB = 2048
_NSTEPS = (VOCAB + _VB - 1) // _VB


def _mm_body(xi_ref, xj_ref, w1t_ref, b1_ref, w2t_ref, b2_ref,
             o1_ref, o2_ref):
    dn = (((0,), (1,)), ((), ()))
    dn1 = (((0,), (0,)), ((), ()))
    ones = jnp.ones((1, BATCH), dtype=jnp.float32)
    b1c = lax.dot_general(b1_ref[...].reshape(1, _VB), ones, dn1,
                          preferred_element_type=jnp.float32)
    b2c = lax.dot_general(b2_ref[...].reshape(1, _VB), ones, dn1,
                          preferred_element_type=jnp.float32)
    o1_ref[...] = lax.dot_general(
        w1t_ref[...], xi_ref[...], dn,
        preferred_element_type=jnp.float32) + b1c
    o2_ref[...] = lax.dot_general(
        w2t_ref[...], xj_ref[...], dn,
        preferred_element_type=jnp.float32) + b2c


def _tc_matmuls(rows, W1t, b1, W2t, b2):
    return pl.pallas_call(
        _mm_body,
        grid=(_NSTEPS,),
        in_specs=[
            pl.BlockSpec((BATCH, DIM), lambda v: (0, 0)),
            pl.BlockSpec((BATCH, DIM), lambda v: (1, 0)),
            pl.BlockSpec((DIM, _VB), lambda v: (0, v)),
            pl.BlockSpec((_VB,), lambda v: (v,)),
            pl.BlockSpec((DIM, _VB), lambda v: (0, v)),
            pl.BlockSpec((_VB,), lambda v: (v,)),
        ],
        out_specs=[
            pl.BlockSpec((_VB, BATCH), lambda v: (v, 0)),
            pl.BlockSpec((_VB, BATCH), lambda v: (v, 0)),
        ],
        out_shape=[
            jax.ShapeDtypeStruct((VOCAB, BATCH), jnp.float32),
            jax.ShapeDtypeStruct((VOCAB, BATCH), jnp.float32),
        ],
        compiler_params=pltpu.CompilerParams(
            dimension_semantics=("parallel",),
        ),
    )(rows, rows, W1t, b1, W2t, b2)


def kernel(i_indices, j_indices, emb, W1, b1, W2, b2):
    rows = _sc_gather(
        emb, i_indices.astype(jnp.int32), j_indices.astype(jnp.int32)
    )
    o1T, o2T = _tc_matmuls(rows, W1.T, b1, W2.T, b2)
    return (o1T.T, o2T.T)
